# Initial kernel scaffold; baseline (speedup 1.0000x reference)
import jax
import jax.numpy as jnp
from jax.experimental import pallas as pl


def kernel(x, edge_index, enc0_Wl, enc0_Wr, enc0_att, enc0_b, enc1_Wl, enc1_Wr, enc1_att, enc1_b, dec0_Wl, dec0_Wr, dec0_att, dec0_b, dec1_Wl, dec1_Wr, dec1_att, dec1_b, mu_W, mu_b, ls_W, ls_b, out_W, out_b):
    raise NotImplementedError("write your pallas kernel here")


# speedup vs baseline: 1.1114x; 1.1114x over previous
import functools

import jax
import jax.numpy as jnp
from jax import lax
from jax.experimental import pallas as pl

N_NODES = 10000
D_FEAT = 128


def _gelu(t):
    return 0.5 * t * (1.0 + lax.erf(t * 0.7071067811865476))


def _proj_body(h_ref, w_ref, b_ref, o_ref, *, preact):
    h = h_ref[...]
    if preact:
        h = _gelu(h + b_ref[...])
    o_ref[...] = jnp.dot(h, w_ref[...], preferred_element_type=jnp.float32)


def _proj(h, w, b_prev, preact):
    n, c = h.shape
    k = w.shape[1]
    b2 = b_prev.reshape(1, c) if preact else jnp.zeros((1, c), jnp.float32)
    body = functools.partial(_proj_body, preact=preact)
    return pl.pallas_call(
        body,
        out_shape=jax.ShapeDtypeStruct((n, k), jnp.float32),
    )(h, w, b2)


def _head_body(a_ref, w_ref, b_ref, mb_ref, lb_ref, eps_ref,
               mu_ref, ls_ref, z_ref):
    h = _gelu(a_ref[...] + b_ref[...])
    mls = jnp.dot(h, w_ref[...], preferred_element_type=jnp.float32)
    mu = mls[:, :64] + mb_ref[...]
    ls = jnp.minimum(mls[:, 64:] + lb_ref[...], 10.0)
    mu_ref[...] = mu
    ls_ref[...] = ls
    z_ref[...] = mu + eps_ref[...] * jnp.exp(ls)


def _vae_head(a1, enc1_b, mu_W, mu_b, ls_W, ls_b, eps):
    n = a1.shape[0]
    w = jnp.concatenate([mu_W, ls_W], axis=1)
    out_shapes = (
        jax.ShapeDtypeStruct((n, 64), jnp.float32),
        jax.ShapeDtypeStruct((n, 64), jnp.float32),
        jax.ShapeDtypeStruct((n, 64), jnp.float32),
    )
    return pl.pallas_call(
        _head_body,
        out_shape=out_shapes,
    )(a1, w, enc1_b.reshape(1, -1), mu_b.reshape(1, -1),
      ls_b.reshape(1, -1), eps)


def _xrec_body(a_ref, w_ref, b_ref, ob_ref, o_ref):
    h = _gelu(a_ref[...] + b_ref[...])
    o_ref[...] = jnp.dot(h, w_ref[...],
                         preferred_element_type=jnp.float32) + ob_ref[...]


def _xrec(a3, dec1_b, out_W, out_b):
    n = a3.shape[0]
    return pl.pallas_call(
        _xrec_body,
        out_shape=jax.ShapeDtypeStruct((n, out_W.shape[1]), jnp.float32),
    )(a3, out_W, dec1_b.reshape(1, -1), out_b.reshape(1, -1))


def _attention(xl, xr, att, src, dst, n):
    e = jax.nn.leaky_relu(xl[src] + xr[dst], 0.2) @ att
    m = jax.ops.segment_max(e, dst, num_segments=n)
    ex = jnp.exp(e - m[dst])
    den = jax.ops.segment_sum(ex, dst, num_segments=n)
    alpha = ex / (den[dst] + 1e-16)
    return jax.ops.segment_sum(alpha[:, None] * xl[src], dst, num_segments=n)


def kernel(x, edge_index,
           enc0_Wl, enc0_Wr, enc0_att, enc0_b,
           enc1_Wl, enc1_Wr, enc1_att, enc1_b,
           dec0_Wl, dec0_Wr, dec0_att, dec0_b,
           dec1_Wl, dec1_Wr, dec1_att, dec1_b,
           mu_W, mu_b, ls_W, ls_b, out_W, out_b):
    n = x.shape[0]
    loop = jnp.arange(n, dtype=edge_index.dtype)
    src = jnp.concatenate([edge_index[0], loop])
    dst = jnp.concatenate([edge_index[1], loop])

    def layer(h, Wl, Wr, att, b_prev, preact):
        w = jnp.concatenate([Wl, Wr], axis=1)
        proj = _proj(h, w, b_prev, preact)
        xl = proj[:, :128]
        xr = proj[:, 128:]
        return _attention(xl, xr, att, src, dst, n)

    a0 = layer(x, enc0_Wl, enc0_Wr, enc0_att, enc0_b, preact=False)
    a1 = layer(a0, enc1_Wl, enc1_Wr, enc1_att, enc0_b, preact=True)
    eps = jax.random.normal(jax.random.key(42), (n, 64), jnp.float32)
    mu, logstd, z = _vae_head(a1, enc1_b, mu_W, mu_b, ls_W, ls_b, eps)
    a2 = layer(z, dec0_Wl, dec0_Wr, dec0_att, dec0_b, preact=False)
    a3 = layer(a2, dec1_Wl, dec1_Wr, dec1_att, dec0_b, preact=True)
    x_rec = _xrec(a3, dec1_b, out_W, out_b)
    return (x_rec, z, mu, logstd)
